# Initial kernel scaffold; baseline (speedup 1.0000x reference)
import jax
import jax.numpy as jnp
from jax.experimental import pallas as pl


def kernel(features, edge_index, edge_weight, W, bias):
    raise NotImplementedError("write your pallas kernel here")


# speedup vs baseline: 7.2630x; 7.2630x over previous
import functools

import jax
import jax.numpy as jnp
from jax import lax
from jax.experimental import pallas as pl
from jax.experimental.pallas import tpu as pltpu
from jax.experimental.pallas import tpu_sc as plsc

N = 10000
E = 320000
D = 128
NC = 2
NS = 16
NW = NC * NS
CHUNK = 128
EPW = ((E + NW * CHUNK - 1) // (NW * CHUNK)) * CHUNK
CHUNKS = EPW // CHUNK
E_PAD = NW * EPW
N_PAD = 10240
ROWS_PER_TILE = N_PAD // NS


def _sc_body(feat_hbm, src_hbm, dst_hbm, w_hbm, zeros_hbm, out_hbm,
             src_v, dst_v, w_v, rows_v, acc_sh, sem):
    cid = lax.axis_index("c")
    sid = lax.axis_index("s")
    wid = sid * NC + cid

    slab = pl.ds(sid * ROWS_PER_TILE, ROWS_PER_TILE)
    pltpu.sync_copy(zeros_hbm.at[slab], acc_sh.at[slab])

    pltpu.sync_copy(src_hbm.at[wid], src_v)
    pltpu.sync_copy(dst_hbm.at[wid], dst_v)
    pltpu.sync_copy(w_hbm.at[wid], w_v)
    plsc.subcore_barrier()

    def chunk_body(j, carry):
        pltpu.async_copy(feat_hbm.at[src_v.at[j]], rows_v, sem).wait()

        def group_body(g, c):
            w_vec = w_v[j, pl.ds(g * 16, 16)]
            for rr in range(16):
                row = g * 16 + rr
                ws = w_vec[rr]
                for c8 in range(D // 16):
                    sl = pl.ds(c8 * 16, 16)
                    rows_v[row, sl] = rows_v[row, sl] * ws
            return c

        lax.fori_loop(0, CHUNK // 16, group_body, 0, unroll=False)

        pltpu.sync_copy(rows_v, acc_sh.at[dst_v.at[j]], add=True)
        return carry

    lax.fori_loop(0, CHUNKS, chunk_body, 0, unroll=False)
    plsc.subcore_barrier()

    pltpu.sync_copy(acc_sh.at[slab], out_hbm.at[cid, slab])


def _make_sc_kernel():
    mesh = plsc.VectorSubcoreMesh(core_axis_name="c", subcore_axis_name="s")
    return pl.kernel(
        _sc_body,
        out_type=jax.ShapeDtypeStruct((NC, N_PAD, D), jnp.float32),
        mesh=mesh,
        scratch_types=[
            pltpu.VMEM((CHUNKS, CHUNK), jnp.int32),
            pltpu.VMEM((CHUNKS, CHUNK), jnp.int32),
            pltpu.VMEM((CHUNKS, CHUNK), jnp.float32),
            pltpu.VMEM((CHUNK, D), jnp.float32),
            pltpu.VMEM_SHARED((N_PAD, D), jnp.float32),
            pltpu.SemaphoreType.DMA,
        ],
    )


def _mm_body(p_ref, w_ref, b_ref, o_ref):
    x = p_ref[0] + p_ref[1]
    o_ref[...] = (
        jnp.dot(x, w_ref[...], preferred_element_type=jnp.float32) + b_ref[...]
    )


MM_BLOCK = 400


def _make_mm_kernel():
    return pl.pallas_call(
        _mm_body,
        grid=(N // MM_BLOCK,),
        in_specs=[
            pl.BlockSpec((NC, MM_BLOCK, D), lambda i: (0, i, 0)),
            pl.BlockSpec((D, D), lambda i: (0, 0)),
            pl.BlockSpec((1, D), lambda i: (0, 0)),
        ],
        out_specs=pl.BlockSpec((MM_BLOCK, D), lambda i: (i, 0)),
        out_shape=jax.ShapeDtypeStruct((N, D), jnp.float32),
    )


def kernel(features, edge_index, edge_weight, W, bias):
    src = edge_index[0]
    dst = edge_index[1]

    pad = E_PAD - E
    if pad:
        fill = (jnp.arange(pad, dtype=jnp.int32) * 37) % N
        src = jnp.concatenate([src, fill])
        dst = jnp.concatenate([dst, fill])
        edge_weight = jnp.concatenate(
            [edge_weight, jnp.zeros((pad,), jnp.float32)]
        )

    src_r = src.reshape(NW, CHUNKS, CHUNK)
    dst_r = dst.reshape(NW, CHUNKS, CHUNK)
    w_r = edge_weight.reshape(NW, CHUNKS, CHUNK)
    zeros = jnp.zeros((N_PAD, D), jnp.float32)

    partials = _make_sc_kernel()(features, src_r, dst_r, w_r, zeros)
    out = _make_mm_kernel()(partials, W, bias.reshape(1, D))
    return out


# speedup vs baseline: 10.6341x; 1.4641x over previous
import functools

import jax
import jax.numpy as jnp
from jax import lax
from jax.experimental import pallas as pl
from jax.experimental.pallas import tpu as pltpu
from jax.experimental.pallas import tpu_sc as plsc

N = 10000
E = 320000
D = 128
NC = 2
NS = 16
NW = NC * NS
CHUNK = 32
EPW = ((E + NW * 128 - 1) // (NW * 128)) * 128
CHUNKS = EPW // CHUNK
E_PAD = NW * EPW
N_PAD = 10240
ROWS_PER_TILE = N_PAD // NS


NBUF = 4


def _sc_body(feat_hbm, sd_hbm, w_hbm, zeros_hbm, out_hbm,
             sd_v, w_v,
             rows0, rows1, rows2, rows3,
             srcb0, srcb1, srcb2, srcb3,
             dstb0, dstb1, dstb2, dstb3,
             gsem0, gsem1, gsem2, gsem3,
             ssem0, ssem1, ssem2, ssem3,
             acc_sh):
    rows = (rows0, rows1, rows2, rows3)
    srcb = (srcb0, srcb1, srcb2, srcb3)
    dstb = (dstb0, dstb1, dstb2, dstb3)
    gsem = (gsem0, gsem1, gsem2, gsem3)
    ssem = (ssem0, ssem1, ssem2, ssem3)

    cid = lax.axis_index("c")
    sid = lax.axis_index("s")
    wid = sid * NC + cid

    slab = pl.ds(sid * ROWS_PER_TILE, ROWS_PER_TILE)
    pltpu.sync_copy(zeros_hbm.at[slab], acc_sh.at[slab])

    pltpu.sync_copy(sd_hbm.at[wid], sd_v)
    pltpu.sync_copy(w_hbm.at[wid], w_v)
    plsc.subcore_barrier()

    def unpack(j, b):
        for g in range(CHUNK // 16):
            off = j * CHUNK + g * 16
            sl = pl.ds(g * 16, 16)
            v = sd_v[off // 128, pl.ds(off % 128, 16)]
            srcb[b][sl] = v & 0x3FFF
            dstb[b][sl] = v >> 14

    def gstart(j, b):
        pltpu.async_copy(feat_hbm.at[srcb[b]], rows[b], gsem[b])

    def gwait(j, b):
        pltpu.make_async_copy(feat_hbm.at[srcb[b]], rows[b], gsem[b]).wait()

    def cstart(j, b):
        pltpu.async_copy(rows[b], acc_sh.at[dstb[b]], ssem[b], add=True)

    def cwait(j, b):
        pltpu.make_async_copy(rows[b], acc_sh.at[dstb[b]], ssem[b]).wait()

    def scale(j, b):
        def group_body(g, c):
            off = j * CHUNK + g * 16
            w_vec = w_v[off // 128, pl.ds(off % 128, 16)]
            for rr in range(16):
                row = g * 16 + rr
                ws = w_vec[rr]
                for c8 in range(D // 16):
                    sl = pl.ds(c8 * 16, 16)
                    rows[b][row, sl] = rows[b][row, sl] * ws
            return c

        lax.fori_loop(0, CHUNK // 16, group_body, 0, unroll=False)

    for b in range(NBUF - 1):
        unpack(b, b)
        gstart(b, b)

    def pipe_body(jj, carry):
        for b in range(NBUF):
            j = jj * NBUF + b
            gwait(j, b)
            bn = (b + NBUF - 1) % NBUF
            jn = j + NBUF - 1
            if b == 0:
                @pl.when(jj > 0)
                def _():
                    cwait(j - 1, bn)

                unpack(jn, bn)
                gstart(jn, bn)
            else:
                cwait(j - 1, bn)

                @pl.when(jn < CHUNKS)
                def _():
                    unpack(jn, bn)
                    gstart(jn, bn)
            scale(j, b)
            cstart(j, b)
        return carry

    lax.fori_loop(0, CHUNKS // NBUF, pipe_body, 0, unroll=False)
    cwait(CHUNKS - 1, NBUF - 1)
    plsc.subcore_barrier()

    pltpu.sync_copy(acc_sh.at[slab], out_hbm.at[cid, slab])


def _make_sc_kernel():
    mesh = plsc.VectorSubcoreMesh(core_axis_name="c", subcore_axis_name="s")
    return pl.kernel(
        _sc_body,
        out_type=jax.ShapeDtypeStruct((NC, N_PAD, D), jnp.float32),
        mesh=mesh,
        scratch_types=[
            pltpu.VMEM((EPW // 128, 128), jnp.int32),
            pltpu.VMEM((EPW // 128, 128), jnp.float32),
        ]
        + [pltpu.VMEM((CHUNK, D), jnp.float32) for _ in range(NBUF)]
        + [pltpu.VMEM((CHUNK,), jnp.int32) for _ in range(2 * NBUF)]
        + [pltpu.SemaphoreType.DMA for _ in range(2 * NBUF)]
        + [
            pltpu.VMEM_SHARED((N_PAD, D), jnp.float32),
        ],
    )


def _mm_body(p_ref, w_ref, b_ref, o_ref):
    x = p_ref[0] + p_ref[1]
    o_ref[...] = (
        jnp.dot(x, w_ref[...], preferred_element_type=jnp.float32) + b_ref[...]
    )


MM_BLOCK = 400


def _make_mm_kernel():
    return pl.pallas_call(
        _mm_body,
        grid=(N // MM_BLOCK,),
        in_specs=[
            pl.BlockSpec((NC, MM_BLOCK, D), lambda i: (0, i, 0)),
            pl.BlockSpec((D, D), lambda i: (0, 0)),
            pl.BlockSpec((1, D), lambda i: (0, 0)),
        ],
        out_specs=pl.BlockSpec((MM_BLOCK, D), lambda i: (i, 0)),
        out_shape=jax.ShapeDtypeStruct((N, D), jnp.float32),
    )


def kernel(features, edge_index, edge_weight, W, bias):
    src = edge_index[0]
    dst = edge_index[1]

    pad = E_PAD - E
    if pad:
        fill = (jnp.arange(pad, dtype=jnp.int32) * 37) % N
        src = jnp.concatenate([src, fill])
        dst = jnp.concatenate([dst, fill])
        edge_weight = jnp.concatenate(
            [edge_weight, jnp.zeros((pad,), jnp.float32)]
        )

    sd = (dst << 14) | src
    sd_r = sd.reshape(NW, EPW // 128, 128)
    w_r = edge_weight.reshape(NW, EPW // 128, 128)
    zeros = jnp.zeros((N_PAD, D), jnp.float32)

    partials = _make_sc_kernel()(features, sd_r, w_r, zeros)
    out = _make_mm_kernel()(partials, W, bias.reshape(1, D))
    return out


# speedup vs baseline: 11.1809x; 1.0514x over previous
import functools

import jax
import jax.numpy as jnp
from jax import lax
from jax.experimental import pallas as pl
from jax.experimental.pallas import tpu as pltpu
from jax.experimental.pallas import tpu_sc as plsc

N = 10000
E = 320000
D = 128
NC = 2
NS = 16
NW = NC * NS
CHUNK = 32
EPW = ((E + NW * 128 - 1) // (NW * 128)) * 128
CHUNKS = EPW // CHUNK
E_PAD = NW * EPW
N_PAD = 10240
ROWS_PER_TILE = N_PAD // NS


NBUF = 4


def _sc_body(feat_hbm, sd_hbm, w_hbm, zeros_hbm, out_hbm,
             sd_v, w_v,
             rows0, rows1, rows2, rows3,
             srcb0, srcb1, srcb2, srcb3,
             dstb0, dstb1, dstb2, dstb3,
             gsem0, gsem1, gsem2, gsem3,
             ssem0, ssem1, ssem2, ssem3,
             acc_sh):
    rows = (rows0, rows1, rows2, rows3)
    srcb = (srcb0, srcb1, srcb2, srcb3)
    dstb = (dstb0, dstb1, dstb2, dstb3)
    gsem = (gsem0, gsem1, gsem2, gsem3)
    ssem = (ssem0, ssem1, ssem2, ssem3)

    cid = lax.axis_index("c")
    sid = lax.axis_index("s")
    wid = sid * NC + cid

    slab = pl.ds(sid * ROWS_PER_TILE, ROWS_PER_TILE)
    pltpu.sync_copy(zeros_hbm.at[slab], acc_sh.at[slab])

    pltpu.sync_copy(sd_hbm.at[wid], sd_v)
    pltpu.sync_copy(w_hbm.at[wid], w_v)
    plsc.subcore_barrier()

    def unpack(j, b):
        for g in range(CHUNK // 16):
            off = j * CHUNK + g * 16
            sl = pl.ds(g * 16, 16)
            v = sd_v[off // 128, pl.ds(off % 128, 16)]
            srcb[b][sl] = v & 0x3FFF
            dstb[b][sl] = v >> 14

    def gstart(j, b):
        pltpu.async_copy(feat_hbm.at[srcb[b]], rows[b], gsem[b])

    def gwait(j, b):
        pltpu.make_async_copy(feat_hbm.at[srcb[b]], rows[b], gsem[b]).wait()

    def cstart(j, b):
        pltpu.async_copy(rows[b], acc_sh.at[dstb[b]], ssem[b], add=True)

    def cwait(j, b):
        pltpu.make_async_copy(rows[b], acc_sh.at[dstb[b]], ssem[b]).wait()

    def scale(j, b):
        def group_body(g, c):
            off = j * CHUNK + g * 16
            w_vec = w_v[off // 128, pl.ds(off % 128, 16)]
            for rr in range(16):
                row = g * 16 + rr
                ws = w_vec[rr]
                for c8 in range(D // 16):
                    sl = pl.ds(c8 * 16, 16)
                    rows[b][row, sl] = rows[b][row, sl] * ws
            return c

        lax.fori_loop(0, CHUNK // 16, group_body, 0, unroll=False)

    for b in range(NBUF - 1):
        unpack(b, b)
        gstart(b, b)

    def pipe_body(jj, carry):
        for b in range(NBUF):
            j = jj * NBUF + b
            gwait(j, b)
            scale(j, b)
            cstart(j, b)
            bn = (b + NBUF - 1) % NBUF
            jn = j + NBUF - 1
            if b == 0:
                @pl.when(jj > 0)
                def _():
                    cwait(j - 1, bn)

                unpack(jn, bn)
                gstart(jn, bn)
            else:
                cwait(j - 1, bn)

                @pl.when(jn < CHUNKS)
                def _():
                    unpack(jn, bn)
                    gstart(jn, bn)
        return carry

    lax.fori_loop(0, CHUNKS // NBUF, pipe_body, 0, unroll=False)
    cwait(CHUNKS - 1, NBUF - 1)
    plsc.subcore_barrier()

    pltpu.sync_copy(acc_sh.at[slab], out_hbm.at[cid, slab])


def _make_sc_kernel():
    mesh = plsc.VectorSubcoreMesh(core_axis_name="c", subcore_axis_name="s")
    return pl.kernel(
        _sc_body,
        out_type=jax.ShapeDtypeStruct((NC, N_PAD, D), jnp.float32),
        mesh=mesh,
        scratch_types=[
            pltpu.VMEM((EPW // 128, 128), jnp.int32),
            pltpu.VMEM((EPW // 128, 128), jnp.float32),
        ]
        + [pltpu.VMEM((CHUNK, D), jnp.float32) for _ in range(NBUF)]
        + [pltpu.VMEM((CHUNK,), jnp.int32) for _ in range(2 * NBUF)]
        + [pltpu.SemaphoreType.DMA for _ in range(2 * NBUF)]
        + [
            pltpu.VMEM_SHARED((N_PAD, D), jnp.float32),
        ],
    )


def _mm_body(p_ref, w_ref, b_ref, o_ref):
    x = p_ref[0] + p_ref[1]
    o_ref[...] = (
        jnp.dot(x, w_ref[...], preferred_element_type=jnp.float32) + b_ref[...]
    )


MM_BLOCK = 400


def _make_mm_kernel():
    return pl.pallas_call(
        _mm_body,
        grid=(N // MM_BLOCK,),
        in_specs=[
            pl.BlockSpec((NC, MM_BLOCK, D), lambda i: (0, i, 0)),
            pl.BlockSpec((D, D), lambda i: (0, 0)),
            pl.BlockSpec((1, D), lambda i: (0, 0)),
        ],
        out_specs=pl.BlockSpec((MM_BLOCK, D), lambda i: (i, 0)),
        out_shape=jax.ShapeDtypeStruct((N, D), jnp.float32),
    )


def kernel(features, edge_index, edge_weight, W, bias):
    src = edge_index[0]
    dst = edge_index[1]

    pad = E_PAD - E
    if pad:
        fill = (jnp.arange(pad, dtype=jnp.int32) * 37) % N
        src = jnp.concatenate([src, fill])
        dst = jnp.concatenate([dst, fill])
        edge_weight = jnp.concatenate(
            [edge_weight, jnp.zeros((pad,), jnp.float32)]
        )

    sd = (dst << 14) | src
    sd_r = sd.reshape(NW, EPW // 128, 128)
    w_r = edge_weight.reshape(NW, EPW // 128, 128)
    zeros = jnp.zeros((N_PAD, D), jnp.float32)

    partials = _make_sc_kernel()(features, sd_r, w_r, zeros)
    out = _make_mm_kernel()(partials, W, bias.reshape(1, D))
    return out
